# baseline (speedup 1.0000x reference)
import functools

import jax
import jax.numpy as jnp
from jax.experimental import pallas as pl

F32 = jnp.float32
BF16 = jnp.bfloat16

TB = 256
B = 1024


def _mm(a, b, dims):
    return jax.lax.dot_general(a.astype(BF16), b.astype(BF16), (dims, ((), ())),
                               preferred_element_type=F32)


def _lin(x, w, b):
    return _mm(x, w, ((1,), (0,))) + b


def _ln(x, g, b):
    m = jnp.mean(x, axis=-1, keepdims=True)
    v = jnp.var(x, axis=-1, keepdims=True)
    return (x - m) / jnp.sqrt(v + 1e-5) * g + b


_SQRT_HALF = 0.7071067811865476


def _gelu(x):
    return 0.5 * x * (1.0 + jax.lax.erf(x * _SQRT_HALF))


def _argmin_rows(d2):
    k = d2.shape[1]
    mn = jnp.min(d2, axis=-1, keepdims=True)
    ii = jax.lax.broadcasted_iota(jnp.int32, d2.shape, 1)
    cand = jnp.where(d2 == mn, ii, k)
    return jnp.min(cand, axis=-1)


def _enc_body(n_res, *refs):
    it = iter(refs)
    h0_ref = next(it)
    g0, bb0 = next(it), next(it)
    res = [[next(it) for _ in range(6)] for _ in range(n_res)]
    wlo, blo, cb_ref = next(it), next(it), next(it)
    z_ref, idx_ref = next(it), next(it)

    h = h0_ref[...]
    del g0, bb0
    for (w1, b1, w2, b2, g, bb) in res:
        t = _lin(h, w1[...], b1[...])
        t = _gelu(t)
        t = _lin(t, w2[...], b2[...])
        h = _ln(h + t, g[...], bb[...])
    z = _lin(h, wlo[...], blo[...])
    z_ref[...] = z

    cb = cb_ref[...]
    zz = jnp.sum(z * z, axis=1, keepdims=True)
    zc = _mm(z, cb, ((1,), (1,)))
    cc = jnp.sum(cb * cb, axis=1)[None, :]
    d2 = zz - 2.0 * zc + cc
    idx_ref[...] = _argmin_rows(d2)[:, None]


def _full_spec(a):
    shape = a.shape
    return pl.BlockSpec(shape, lambda i: (0,) * len(shape))


def _res_params_list(p, n_res):
    out = []
    for r in p["res"][:n_res]:
        out += [r["l1"]["w"], r["l1"]["b"][None, :], r["l2"]["w"], r["l2"]["b"][None, :],
                r["ln"]["g"][None, :], r["ln"]["b"][None, :]]
    return out


def _encoder_vq(h0, p, cb, n_res):
    hdim = h0.shape[1]
    params = ([p["ln0"]["g"][None, :], p["ln0"]["b"][None, :]]
              + _res_params_list(p, n_res)
              + [p["lo"]["w"], p["lo"]["b"][None, :], cb])
    in_specs = [pl.BlockSpec((TB, hdim), lambda i: (i, 0))] + [_full_spec(a) for a in params]
    out_specs = [pl.BlockSpec((TB, 128), lambda i: (i, 0)),
                 pl.BlockSpec((TB, 1), lambda i: (i, 0))]
    z, idx = pl.pallas_call(
        functools.partial(_enc_body, n_res),
        grid=(B // TB,),
        in_specs=in_specs,
        out_specs=out_specs,
        out_shape=[jax.ShapeDtypeStruct((B, 128), F32),
                   jax.ShapeDtypeStruct((B, 1), jnp.int32)],
    )(h0, *params)
    return z, idx[:, 0]


def _exact_gather(cb, idx):
    k = cb.shape[0]
    u = jax.lax.bitcast_convert_type(cb, jnp.uint32)
    onehot = (idx == jax.lax.broadcasted_iota(jnp.int32, (idx.shape[0], k), 1))
    onehot = onehot.astype(BF16)
    acc = jnp.zeros((idx.shape[0], cb.shape[1]), dtype=jnp.uint32)
    for shift in (0, 8, 16, 24):
        byte = ((u >> shift) & 0xFF).astype(F32)
        g = jax.lax.dot_general(onehot, byte.astype(BF16), ((((1,), (0,))), ((), ())),
                                preferred_element_type=F32)
        acc = acc | (g.astype(jnp.uint32) << shift)
    return jax.lax.bitcast_convert_type(acc, F32)


def _dec_front_body(*refs):
    it = iter(refs)
    ip_ref, im_ref, id_ref, cbp_ref, cbm_ref, cbd_ref = (next(it) for _ in range(6))
    w0, b0, g0, bb0 = (next(it) for _ in range(4))
    res = [[next(it) for _ in range(6)] for _ in range(2)]
    h_ref = next(it)

    zq = jnp.concatenate([
        _exact_gather(cbp_ref[...], ip_ref[...]),
        _exact_gather(cbm_ref[...], im_ref[...]),
        _exact_gather(cbd_ref[...], id_ref[...]),
    ], axis=-1)
    h = _lin(zq, w0[...], b0[...])
    h = _ln(h, g0[...], bb0[...])
    h = _gelu(h)
    for (w1, b1, w2, b2, g, bb) in res:
        t = _lin(h, w1[...], b1[...])
        t = _gelu(t)
        t = _lin(t, w2[...], b2[...])
        h = _ln(h + t, g[...], bb[...])
    h_ref[...] = h


def _dec_front(i_p, i_m, i_d, cbs, dp):
    params = ([dp["l0"]["w"], dp["l0"]["b"][None, :],
               dp["ln0"]["g"][None, :], dp["ln0"]["b"][None, :]]
              + _res_params_list(dp, 2))
    idx_spec = pl.BlockSpec((TB, 1), lambda i: (i, 0))
    in_specs = ([idx_spec] * 3 + [_full_spec(a) for a in cbs]
                + [_full_spec(a) for a in params])
    return pl.pallas_call(
        _dec_front_body,
        grid=(B // TB,),
        in_specs=in_specs,
        out_specs=pl.BlockSpec((TB, 512), lambda i: (i, 0)),
        out_shape=jax.ShapeDtypeStruct((B, 512), F32),
    )(i_p[:, None], i_m[:, None], i_d[:, None], *cbs, *params)


def _dec_out_body(h_ref, w_ref, b_ref, o_ref):
    o_ref[...] = _lin(h_ref[...], w_ref[...], b_ref[...])


NPAD = 10240
NC = 4


def _dec_out(h, wlo, blo):
    n = wlo.shape[1]
    wp = jnp.pad(wlo, ((0, 0), (0, NPAD - n)))
    bp = jnp.pad(blo, ((0, NPAD - n),))[None, :]
    out = pl.pallas_call(
        _dec_out_body,
        grid=(NC, B // TB),
        in_specs=[pl.BlockSpec((TB, 512), lambda c, i: (i, 0)),
                  pl.BlockSpec((512, NPAD // NC), lambda c, i: (0, c)),
                  pl.BlockSpec((1, NPAD // NC), lambda c, i: (0, c))],
        out_specs=pl.BlockSpec((TB, NPAD // NC), lambda c, i: (i, c)),
        out_shape=jax.ShapeDtypeStruct((B, NPAD), F32),
    )(h, wp, bp)
    return out[:, :n]


def _enc_prefix(x, p):
    h = x @ p["l0"]["w"] + p["l0"]["b"]
    m = jnp.mean(h, axis=-1, keepdims=True)
    v = jnp.var(h, axis=-1, keepdims=True)
    h = (h - m) / jnp.sqrt(v + 1e-5) * p["ln0"]["g"] + p["ln0"]["b"]
    return jax.nn.gelu(h, approximate=False)


def kernel(pose, motion, dynamics, params):
    p = params
    h0_p = _enc_prefix(pose, p["pose_enc"])
    h0_m = _enc_prefix(motion, p["motion_enc"])
    h0_d = _enc_prefix(dynamics, p["dyn_enc"])
    z_p, i_p = _encoder_vq(h0_p, p["pose_enc"], p["pose_cb"], 2)
    z_m, i_m = _encoder_vq(h0_m, p["motion_enc"], p["motion_cb"], 1)
    z_d, i_d = _encoder_vq(h0_d, p["dyn_enc"], p["dyn_cb"], 1)
    h = _dec_front(i_p, i_m, i_d, [p["pose_cb"], p["motion_cb"], p["dyn_cb"]],
                   p["decoder"])
    recon = _dec_out(h, p["decoder"]["lo"]["w"], p["decoder"]["lo"]["b"])
    return recon, i_p, i_m, i_d
